# baseline (speedup 1.0000x reference)
import jax
import jax.numpy as jnp
from jax import lax
from jax.experimental import pallas as pl
from jax.experimental.pallas import tpu as pltpu
from jax.experimental.pallas import tpu_sc as plsc

_B = 16384
_NW = 32
_RW = _B // _NW
_G = 32
_NG = _RW // _G
_XW = 128


def _sc_gather_kernel(
    idx_h,
    pok_h, mv_h, ab_tab_h, it_tab_h, ty_h,
    x_o,
    pok_v, mv_v, ab_v, it_v, ty_v,
    idx_v, xball, isem, wsem,
):
  wid = lax.axis_index("c") * 16 + lax.axis_index("s")
  base = wid * _RW

  th = [pltpu.async_copy(h, v, isem) for h, v in
        ((pok_h, pok_v), (mv_h, mv_v), (ab_tab_h, ab_v), (it_tab_h, it_v),
         (ty_h, ty_v))]
  th += [pltpu.async_copy(idx_h.at[pl.ds(p * _B + base, _RW)],
                          idx_v.at[pl.ds(p * _RW, _RW)], isem)
         for p in range(4)]
  for h in th:
    h.wait()

  gsz = _G * _XW

  def grp_body(g, _):
    obase = (g % 2) * gsz

    @pl.when(g >= 2)
    def _reclaim():
      pltpu.make_async_copy(
          xball.at[pl.ds(0, gsz)],
          x_o.at[pl.ds(base * _XW, gsz)], wsem).wait()

    for sub in range(_G // 16):
      soff = (g * (_G // 16) + sub) * 16
      w0v = idx_v[pl.ds(soff, 16)]
      w1v = idx_v[pl.ds(_RW + soff, 16)]
      w2v = idx_v[pl.ds(2 * _RW + soff, 16)]
      w3v = idx_v[pl.ds(3 * _RW + soff, 16)]
      for j in range(16):
        o = obase + (sub * 16 + j) * _XW
        w0 = w0v[j]
        w1 = w1v[j]
        w2 = w2v[j]
        w3 = w3v[j]
        si = w0 & 0xFFFF
        i0 = lax.shift_right_logical(w0, 16)
        i1 = w1 & 0xFFFF
        i2 = lax.shift_right_logical(w1, 16)
        i3 = w2 & 0xFFFF
        ao = lax.shift_right_logical(w2, 16)
        io = w3 & 0xFFFF
        to = lax.shift_right_logical(w3, 16)
        for c in range(3):
          xball[pl.ds(o + c * 16, 16)] = pok_v[pl.ds(si + c * 16, 16)]
        for c in range(2):
          acc = (mv_v[pl.ds(i0 + c * 16, 16)] + mv_v[pl.ds(i1 + c * 16, 16)]
                 + mv_v[pl.ds(i2 + c * 16, 16)]
                 + mv_v[pl.ds(i3 + c * 16, 16)])
          xball[pl.ds(o + 48 + c * 16, 16)] = acc
        xball[pl.ds(o + 80, 16)] = ab_v[pl.ds(ao, 16)]
        xball[pl.ds(o + 96, 16)] = it_v[pl.ds(io, 16)]
        xball[pl.ds(o + 112, 16)] = ty_v[pl.ds(to, 16)]
    pltpu.async_copy(
        xball.at[pl.ds(obase, gsz)],
        x_o.at[pl.ds((base + g * _G) * _XW, gsz)], wsem)
    return 0

  lax.fori_loop(0, _NG, grp_body, 0)
  for _ in range(2):
    pltpu.make_async_copy(
        xball.at[pl.ds(0, gsz)],
        x_o.at[pl.ds(base * _XW, gsz)], wsem).wait()


def _make_sc_gather():
  f32 = jnp.float32
  i32 = jnp.int32
  out_type = [
      jax.ShapeDtypeStruct((_B * _XW,), f32),
  ]
  scratch = [
      pltpu.VMEM((1025 * 48,), f32),
      pltpu.VMEM((921 * 32,), f32),
      pltpu.VMEM((310 * 16,), f32),
      pltpu.VMEM((1200 * 16,), f32),
      pltpu.VMEM((19 * 16,), f32),
      pltpu.VMEM((4 * _RW,), i32),
      pltpu.VMEM((2 * _G * _XW,), f32),
      pltpu.SemaphoreType.DMA,
      pltpu.SemaphoreType.DMA,
  ]
  mesh = plsc.VectorSubcoreMesh(core_axis_name="c", subcore_axis_name="s")
  return pl.kernel(
      _sc_gather_kernel, out_type=out_type, mesh=mesh,
      scratch_types=scratch,
      compiler_params=pltpu.CompilerParams(use_tc_tiling_on_sc=False,
                                           disable_bounds_checks=True))


_sc_gather = _make_sc_gather()

_BS = 1024


def _tc_mlp_kernel(x, pk, ff, tytab, w1a, w1t2, w1mt, w1ff,
                   b1, w2, b2, out):
  f32 = jnp.float32
  xv = x[...]

  w = pk[...]

  cnt = lax.shift_right_logical(w, 25).astype(f32)
  rm = 1.0 / jnp.maximum(cnt, 1.0)
  cols128 = lax.broadcasted_iota(jnp.int32, (_BS, _XW), 1)
  xs = xv * jnp.where((cols128 >= 48) & (cols128 < 80), rm, 1.0)

  cols = lax.broadcasted_iota(jnp.int32, (_BS, 32), 1)
  t2 = w & 31
  oh2 = (cols == t2).astype(f32)
  ohsum = jnp.zeros((_BS, 32), f32)
  ct = jnp.zeros((_BS, 1), f32)
  for j in range(4):
    c = lax.shift_right_logical(w, 5 + 5 * j) & 31
    nz = c != 0
    ohsum = ohsum + ((cols == c) & nz).astype(f32)
    ct = ct + nz.astype(f32)
  ohs = ohsum * (1.0 / jnp.maximum(ct, 1.0))

  p2 = jnp.dot(tytab[...], w1t2[...], preferred_element_type=f32)
  pt = jnp.dot(tytab[...], w1mt[...], preferred_element_type=f32)
  h = (jnp.dot(xs, w1a[...], preferred_element_type=f32)
       + jnp.dot(oh2, p2, preferred_element_type=f32)
       + jnp.dot(ohs, pt, preferred_element_type=f32)
       + jnp.dot(ff[...], w1ff[...], preferred_element_type=f32)
       + b1[...])
  h = jnp.maximum(h, 0.0)
  out[...] = jnp.maximum(
      jnp.dot(h, w2[...], preferred_element_type=f32) + b2[...], 0.0)


def _make_tc_mlp():
  def bspec(cols):
    return pl.BlockSpec((_BS, cols), lambda i: (i, 0))
  in_specs = [
      bspec(_XW),
      bspec(1),
      bspec(32),
      pl.BlockSpec((32, 16), lambda i: (0, 0)),
      pl.BlockSpec((128, 256), lambda i: (0, 0)),
      pl.BlockSpec((16, 256), lambda i: (0, 0)),
      pl.BlockSpec((16, 256), lambda i: (0, 0)),
      pl.BlockSpec((32, 256), lambda i: (0, 0)),
      pl.BlockSpec((1, 256), lambda i: (0, 0)),
      pl.BlockSpec((256, 128), lambda i: (0, 0)),
      pl.BlockSpec((1, 128), lambda i: (0, 0)),
  ]
  return pl.pallas_call(
      _tc_mlp_kernel,
      grid=(_B // _BS,),
      in_specs=in_specs,
      out_specs=pl.BlockSpec((_BS, 128), lambda i: (i, 0)),
      out_shape=jax.ShapeDtypeStruct((_B, 128), jnp.float32),
  )


_tc_mlp = _make_tc_mlp()


def kernel(species_idx, move_indices, ability_idx, item_idx, type_indices,
           move_type_indices, float_features, pokemon_table, move_table,
           ability_table, item_table, type_table, W1, b1, W2, b2):
  f32 = jnp.float32
  mv_tab = jnp.pad(
      jnp.concatenate([move_table, jnp.zeros((1, 24), f32)], axis=0),
      ((0, 0), (0, 8)))
  ty_pad = jnp.pad(type_table, ((0, 13), (0, 0)))
  w1a = jnp.concatenate([W1[:72], jnp.zeros((8, 256), f32), W1[72:120]],
                        axis=0)
  w1t2 = W1[120:136]
  w1mt = W1[136:152]
  w1ff = jnp.concatenate([W1[152:183], jnp.zeros((1, 256), f32)], axis=0)

  mvo = jnp.where(move_indices == 0, 920, move_indices) * 32
  idx_pk = jnp.concatenate([
      (species_idx * 48 | mvo[:, 0] << 16)[None, :],
      (mvo[:, 1] | mvo[:, 2] << 16)[None, :],
      (mvo[:, 3] | ability_idx * 16 << 16)[None, :],
      (item_idx * 16 | type_indices[:, 0] * 16 << 16)[None, :],
  ], axis=0)

  (x,) = _sc_gather(
      idx_pk.reshape(-1),
      pokemon_table.reshape(-1), mv_tab.reshape(-1),
      ability_table.reshape(-1), item_table.reshape(-1),
      type_table.reshape(-1))

  pk = (type_indices[:, 1]
        | (move_type_indices[:, 0] << 5)
        | (move_type_indices[:, 1] << 10)
        | (move_type_indices[:, 2] << 15)
        | (move_type_indices[:, 3] << 20)
        | (jnp.sum((move_indices != 0).astype(jnp.int32), axis=1) << 25))
  ffp = jnp.pad(float_features, ((0, 0), (0, 1)))

  return _tc_mlp(x.reshape(_B, _XW), pk[:, None], ffp, ty_pad, w1a, w1t2,
                 w1mt, w1ff, b1.reshape(1, 256), W2, b2.reshape(1, 128))


# speedup vs baseline: 1.2608x; 1.2608x over previous
import jax
import jax.numpy as jnp
from jax import lax
from jax.experimental import pallas as pl
from jax.experimental.pallas import tpu as pltpu
from jax.experimental.pallas import tpu_sc as plsc

_B = 16384
_NW = 32
_RW = _B // _NW
_G = 32
_NG = _RW // _G
_XW = 128


def _sc_gather_kernel(
    idx_h,
    pok_h, mv_h, ab_tab_h, it_tab_h, ty_h,
    x_o,
    pok_v, mv_v, ab_v, it_v, ty_v,
    idx_v, xball, isem, wsem,
):
  wid = lax.axis_index("c") * 16 + lax.axis_index("s")
  base = wid * _RW

  th = [pltpu.async_copy(h, v, isem) for h, v in
        ((pok_h, pok_v), (mv_h, mv_v), (ab_tab_h, ab_v), (it_tab_h, it_v),
         (ty_h, ty_v))]
  th += [pltpu.async_copy(idx_h.at[pl.ds(p * _B + base, _RW)],
                          idx_v.at[pl.ds(p * _RW, _RW)], isem)
         for p in range(5)]
  for h in th:
    h.wait()

  def z_body(i, _):
    xball[pl.ds(i * _XW + 112, 16)] = jnp.zeros((16,), jnp.float32)
    return 0

  lax.fori_loop(0, 2 * _G, z_body, 0)

  gsz = _G * _XW
  _iota16 = lax.iota(jnp.int32, 16)

  def grp_body(g, _):
    obase = (g % 2) * gsz

    @pl.when(g >= 2)
    def _reclaim():
      pltpu.make_async_copy(
          xball.at[pl.ds(0, gsz)],
          x_o.at[pl.ds(base * _XW, gsz)], wsem).wait()

    for sub in range(_G // 16):
      soff = (g * (_G // 16) + sub) * 16
      w0v = idx_v[pl.ds(soff, 16)]
      w1v = idx_v[pl.ds(_RW + soff, 16)]
      w2v = idx_v[pl.ds(2 * _RW + soff, 16)]
      w3v = idx_v[pl.ds(3 * _RW + soff, 16)]
      pkv = idx_v[pl.ds(4 * _RW + soff, 16)]
      rb = obase + (sub * 16 + _iota16) * _XW + 120
      plsc.store_scatter(xball, [rb], plsc.bitcast(pkv, jnp.float32))
      for j in range(16):
        o = obase + (sub * 16 + j) * _XW
        w0 = w0v[j]
        w1 = w1v[j]
        w2 = w2v[j]
        w3 = w3v[j]
        si = w0 & 0xFFFF
        i0 = lax.shift_right_logical(w0, 16)
        i1 = w1 & 0xFFFF
        i2 = lax.shift_right_logical(w1, 16)
        i3 = w2 & 0xFFFF
        ao = lax.shift_right_logical(w2, 16)
        io = w3 & 0xFFFF
        to = lax.shift_right_logical(w3, 16)
        for c in range(3):
          xball[pl.ds(o + c * 16, 16)] = pok_v[pl.ds(si + c * 16, 16)]
        for c in range(2):
          acc = (mv_v[pl.ds(i0 + c * 16, 16)] + mv_v[pl.ds(i1 + c * 16, 16)]
                 + mv_v[pl.ds(i2 + c * 16, 16)]
                 + mv_v[pl.ds(i3 + c * 16, 16)])
          xball[pl.ds(o + 48 + c * 16, 16)] = acc
        xball[pl.ds(o + 72, 16)] = ab_v[pl.ds(ao, 16)]
        xball[pl.ds(o + 88, 16)] = it_v[pl.ds(io, 16)]
        xball[pl.ds(o + 104, 16)] = ty_v[pl.ds(to, 16)]
    pltpu.async_copy(
        xball.at[pl.ds(obase, gsz)],
        x_o.at[pl.ds((base + g * _G) * _XW, gsz)], wsem)
    return 0

  lax.fori_loop(0, _NG, grp_body, 0)
  for _ in range(2):
    pltpu.make_async_copy(
        xball.at[pl.ds(0, gsz)],
        x_o.at[pl.ds(base * _XW, gsz)], wsem).wait()


def _make_sc_gather():
  f32 = jnp.float32
  i32 = jnp.int32
  out_type = [
      jax.ShapeDtypeStruct((_B * _XW,), f32),
  ]
  scratch = [
      pltpu.VMEM((1025 * 48,), f32),
      pltpu.VMEM((921 * 32,), f32),
      pltpu.VMEM((310 * 16,), f32),
      pltpu.VMEM((1200 * 16,), f32),
      pltpu.VMEM((19 * 16,), f32),
      pltpu.VMEM((5 * _RW,), i32),
      pltpu.VMEM((2 * _G * _XW,), f32),
      pltpu.SemaphoreType.DMA,
      pltpu.SemaphoreType.DMA,
  ]
  mesh = plsc.VectorSubcoreMesh(core_axis_name="c", subcore_axis_name="s")
  return pl.kernel(
      _sc_gather_kernel, out_type=out_type, mesh=mesh,
      scratch_types=scratch,
      compiler_params=pltpu.CompilerParams(use_tc_tiling_on_sc=False,
                                           disable_bounds_checks=True,
                                           needs_layout_passes=False))


_sc_gather = _make_sc_gather()

_BS = 1024


def _tc_mlp_kernel(x, ff, tytab, w1a, w1t2, w1mt, w1ff,
                   b1, w2, b2, out):
  f32 = jnp.float32
  xv = x[...]

  w = lax.bitcast_convert_type(xv[:, 120:121], jnp.int32)

  cnt = lax.shift_right_logical(w, 25).astype(f32)
  rm = 1.0 / jnp.maximum(cnt, 1.0)
  cols128 = lax.broadcasted_iota(jnp.int32, (_BS, _XW), 1)
  xs = xv * jnp.where((cols128 >= 48) & (cols128 < 72), rm, 1.0)

  cols = lax.broadcasted_iota(jnp.int32, (_BS, 32), 1)
  t2 = w & 31
  oh2 = (cols == t2).astype(f32)
  ohsum = jnp.zeros((_BS, 32), f32)
  ct = jnp.zeros((_BS, 1), f32)
  for j in range(4):
    c = lax.shift_right_logical(w, 5 + 5 * j) & 31
    nz = c != 0
    ohsum = ohsum + ((cols == c) & nz).astype(f32)
    ct = ct + nz.astype(f32)
  ohs = ohsum * (1.0 / jnp.maximum(ct, 1.0))

  p2 = jnp.dot(tytab[...], w1t2[...], preferred_element_type=f32)
  pt = jnp.dot(tytab[...], w1mt[...], preferred_element_type=f32)
  h = (jnp.dot(xs, w1a[...], preferred_element_type=f32)
       + jnp.dot(oh2, p2, preferred_element_type=f32)
       + jnp.dot(ohs, pt, preferred_element_type=f32)
       + jnp.dot(ff[...], w1ff[...], preferred_element_type=f32)
       + b1[...])
  h = jnp.maximum(h, 0.0)
  out[...] = jnp.maximum(
      jnp.dot(h, w2[...], preferred_element_type=f32) + b2[...], 0.0)


def _make_tc_mlp():
  def bspec(cols):
    return pl.BlockSpec((_BS, cols), lambda i: (i, 0))
  in_specs = [
      bspec(_XW),
      bspec(31),
      pl.BlockSpec((32, 16), lambda i: (0, 0)),
      pl.BlockSpec((128, 256), lambda i: (0, 0)),
      pl.BlockSpec((16, 256), lambda i: (0, 0)),
      pl.BlockSpec((16, 256), lambda i: (0, 0)),
      pl.BlockSpec((31, 256), lambda i: (0, 0)),
      pl.BlockSpec((1, 256), lambda i: (0, 0)),
      pl.BlockSpec((256, 128), lambda i: (0, 0)),
      pl.BlockSpec((1, 128), lambda i: (0, 0)),
  ]
  return pl.pallas_call(
      _tc_mlp_kernel,
      grid=(_B // _BS,),
      in_specs=in_specs,
      out_specs=pl.BlockSpec((_BS, 128), lambda i: (i, 0)),
      out_shape=jax.ShapeDtypeStruct((_B, 128), jnp.float32),
  )


_tc_mlp = _make_tc_mlp()


def kernel(species_idx, move_indices, ability_idx, item_idx, type_indices,
           move_type_indices, float_features, pokemon_table, move_table,
           ability_table, item_table, type_table, W1, b1, W2, b2):
  f32 = jnp.float32
  mv_tab = jnp.pad(
      jnp.concatenate([move_table, jnp.zeros((1, 24), f32)], axis=0),
      ((0, 0), (0, 8)))
  ty_pad = jnp.pad(type_table, ((0, 13), (0, 0)))
  w1a = jnp.concatenate([W1[:120], jnp.zeros((8, 256), f32)], axis=0)
  w1t2 = W1[120:136]
  w1mt = W1[136:152]
  w1ff = W1[152:183]

  mvo = jnp.where(move_indices == 0, 920, move_indices) * 32
  pk = (type_indices[:, 1]
        | (move_type_indices[:, 0] << 5)
        | (move_type_indices[:, 1] << 10)
        | (move_type_indices[:, 2] << 15)
        | (move_type_indices[:, 3] << 20)
        | (jnp.sum((move_indices != 0).astype(jnp.int32), axis=1) << 25))
  idx_pk = jnp.concatenate([
      (species_idx * 48 | mvo[:, 0] << 16)[None, :],
      (mvo[:, 1] | mvo[:, 2] << 16)[None, :],
      (mvo[:, 3] | ability_idx * 16 << 16)[None, :],
      (item_idx * 16 | type_indices[:, 0] * 16 << 16)[None, :],
      pk[None, :],
  ], axis=0)

  (x,) = _sc_gather(
      idx_pk.reshape(-1),
      pokemon_table.reshape(-1), mv_tab.reshape(-1),
      ability_table.reshape(-1), item_table.reshape(-1),
      type_table.reshape(-1))

  return _tc_mlp(x.reshape(_B, _XW), float_features, ty_pad, w1a, w1t2,
                 w1mt, w1ff, b1.reshape(1, 256), W2, b2.reshape(1, 128))
